# baseline (speedup 1.0000x reference)
import functools

import jax
import jax.numpy as jnp
from jax import lax
from jax.experimental import pallas as pl
from jax.experimental.pallas import tpu as pltpu
from jax.experimental.pallas import tpu_sc as plsc

CB = 8192
D = 256
NROWS = 16384
BR = 512
BETA = 0.25


def _scores_body(a_ref, z_ref, c_ref, b_ref, codes_ref, loss_ref, ent_ref):
    r = pl.program_id(0)
    zc = jnp.dot(z_ref[...], c_ref[...], preferred_element_type=jnp.float32)
    dist = (a_ref[...] - 2.0 * zc) + b_ref[...]

    m = jnp.min(dist, axis=1, keepdims=True)
    lanes = lax.broadcasted_iota(jnp.int32, dist.shape, 1)
    idx = jnp.min(jnp.where(dist == m, lanes, CB), axis=1)
    codes_ref[...] = idx.reshape(BR, 1)

    e = jnp.exp(m - dist)
    s = jnp.sum(e, axis=1, keepdims=True)
    wd = jnp.sum(e * dist, axis=1, keepdims=True)
    ent = wd / s + (jnp.log(s) - m)

    loss_blk = jnp.sum(m)
    ent_blk = jnp.sum(ent)
    prev_l = jnp.where(r == 0, 0.0, loss_ref[0, 0])
    prev_e = jnp.where(r == 0, 0.0, ent_ref[0, 0])
    loss_ref[0, 0] = prev_l + loss_blk
    ent_ref[0, 0] = prev_e + ent_blk


def _scores_call(a, z_flat, codebook, b, nrows, off):
    grid = (nrows // BR,)
    ob = off // BR
    return pl.pallas_call(
        _scores_body,
        grid=grid,
        in_specs=[
            pl.BlockSpec((BR, 1), lambda r: (r + ob, 0)),
            pl.BlockSpec((BR, D), lambda r: (r + ob, 0)),
            pl.BlockSpec((D, CB), lambda r: (0, 0)),
            pl.BlockSpec((1, CB), lambda r: (0, 0)),
        ],
        out_specs=[
            pl.BlockSpec((BR, 1), lambda r: (r, 0)),
            pl.BlockSpec(memory_space=pltpu.SMEM),
            pl.BlockSpec(memory_space=pltpu.SMEM),
        ],
        out_shape=[
            jax.ShapeDtypeStruct((nrows, 1), jnp.int32),
            jax.ShapeDtypeStruct((1, 1), jnp.float32),
            jax.ShapeDtypeStruct((1, 1), jnp.float32),
        ],
    )(a, z_flat, codebook, b)


_NW = 32
_CHUNK = 128


def _gather_body(per_w, codes_hbm, table_hbm, out_hbm,
                 idx_all, rows0, rows1, gsem0, gsem1, wsem0, wsem1):
    nch = per_w // _CHUNK
    wid = lax.axis_index("s") * 2 + lax.axis_index("c")
    base = wid * per_w
    rows = (rows0, rows1)
    gsems = (gsem0, gsem1)
    wsems = (wsem0, wsem1)

    pltpu.sync_copy(codes_hbm.at[pl.ds(base, per_w)], idx_all)

    gathers = [None, None]
    writes = [None, None]
    gathers[0] = pltpu.async_copy(
        table_hbm.at[idx_all.at[pl.ds(0, _CHUNK)]], rows0, gsem0)
    for j in range(nch):
        cur = j % 2
        nxt = 1 - cur
        if j + 1 < nch:
            if writes[nxt] is not None:
                writes[nxt].wait()
            gathers[nxt] = pltpu.async_copy(
                table_hbm.at[idx_all.at[pl.ds((j + 1) * _CHUNK, _CHUNK)]],
                rows[nxt], gsems[nxt])
        gathers[cur].wait()
        writes[cur] = pltpu.async_copy(
            rows[cur], out_hbm.at[pl.ds(base + j * _CHUNK, _CHUNK)],
            wsems[cur])
    writes[0].wait()
    writes[1].wait()


@functools.cache
def _gather_call(nrows):
    per_w = nrows // _NW
    return pl.kernel(
        functools.partial(_gather_body, per_w),
        out_type=jax.ShapeDtypeStruct((nrows, D), jnp.float32),
        mesh=plsc.VectorSubcoreMesh(core_axis_name="c", subcore_axis_name="s"),
        scratch_types=[
            pltpu.VMEM((per_w,), jnp.int32),
            pltpu.VMEM((_CHUNK, D), jnp.float32),
            pltpu.VMEM((_CHUNK, D), jnp.float32),
            pltpu.SemaphoreType.DMA,
            pltpu.SemaphoreType.DMA,
            pltpu.SemaphoreType.DMA,
            pltpu.SemaphoreType.DMA,
        ],
    )


def kernel(z_e, codebook):
    B, N, _ = z_e.shape
    z_flat = z_e.reshape(-1, D)
    a = jnp.sum(z_flat ** 2, axis=1, keepdims=True)
    b = jnp.sum(codebook ** 2, axis=1, keepdims=True).T

    codes2, loss_sum, ent_sum = _scores_call(a, z_flat, codebook.T, b, NROWS, 0)
    codes = codes2.reshape(-1)

    z_q = _gather_call(NROWS)(codes, codebook)

    z_q_st = z_q.reshape(B, N, D)
    codes_r = codes.reshape(B, N)
    loss_vq = (1.0 + BETA) * loss_sum[0, 0] / float(NROWS * D)
    perp = jnp.exp(ent_sum[0, 0] / float(NROWS))
    return (z_q_st, codes_r, loss_vq, perp)


# speedup vs baseline: 1.0071x; 1.0071x over previous
import functools

import jax
import jax.numpy as jnp
from jax import lax
from jax.experimental import pallas as pl
from jax.experimental.pallas import tpu as pltpu
from jax.experimental.pallas import tpu_sc as plsc

CB = 8192
D = 256
NROWS = 16384
BR = 512
BETA = 0.25


def _scores_body(a_ref, z_ref, c_hbm, b_ref, codes_ref, loss_ref, ent_ref,
                 c_vmem, csem):
    r = pl.program_id(0)

    @pl.when(r == 0)
    def _load_codebook():
        pltpu.async_copy(c_hbm, c_vmem, csem).wait()

    zc = lax.dot_general(z_ref[...], c_vmem[...], (((1,), (1,)), ((), ())),
                         preferred_element_type=jnp.float32)
    dist = (a_ref[...] - 2.0 * zc) + b_ref[...]

    m = jnp.min(dist, axis=1, keepdims=True)
    lanes = lax.broadcasted_iota(jnp.int32, dist.shape, 1)
    idx = jnp.min(jnp.where(dist == m, lanes, CB), axis=1)
    codes_ref[...] = idx.reshape(BR, 1)

    e = jnp.exp(m - dist)
    s = jnp.sum(e, axis=1, keepdims=True)
    wd = jnp.sum(e * dist, axis=1, keepdims=True)
    ent = wd / s + (jnp.log(s) - m)

    loss_blk = jnp.sum(m)
    ent_blk = jnp.sum(ent)
    prev_l = jnp.where(r == 0, 0.0, loss_ref[0, 0])
    prev_e = jnp.where(r == 0, 0.0, ent_ref[0, 0])
    loss_ref[0, 0] = prev_l + loss_blk
    ent_ref[0, 0] = prev_e + ent_blk


def _scores_call(a, z_flat, codebook, b, nrows, off):
    grid = (nrows // BR,)
    ob = off // BR
    return pl.pallas_call(
        _scores_body,
        grid=grid,
        in_specs=[
            pl.BlockSpec((BR, 1), lambda r: (r + ob, 0)),
            pl.BlockSpec((BR, D), lambda r: (r + ob, 0)),
            pl.BlockSpec(memory_space=pltpu.MemorySpace.HBM),
            pl.BlockSpec((1, CB), lambda r: (0, 0)),
        ],
        scratch_shapes=[
            pltpu.VMEM((CB, D), jnp.float32),
            pltpu.SemaphoreType.DMA,
        ],
        out_specs=[
            pl.BlockSpec((BR, 1), lambda r: (r, 0)),
            pl.BlockSpec(memory_space=pltpu.SMEM),
            pl.BlockSpec(memory_space=pltpu.SMEM),
        ],
        out_shape=[
            jax.ShapeDtypeStruct((nrows, 1), jnp.int32),
            jax.ShapeDtypeStruct((1, 1), jnp.float32),
            jax.ShapeDtypeStruct((1, 1), jnp.float32),
        ],
    )(a, z_flat, codebook, b)


_NW = 32
_CHUNK = 128


def _gather_body(per_w, codes_hbm, table_hbm, out_hbm,
                 idx_all, rows0, rows1, gsem0, gsem1, wsem0, wsem1):
    nch = per_w // _CHUNK
    wid = lax.axis_index("s") * 2 + lax.axis_index("c")
    base = wid * per_w
    rows = (rows0, rows1)
    gsems = (gsem0, gsem1)
    wsems = (wsem0, wsem1)

    pltpu.sync_copy(codes_hbm.at[pl.ds(base, per_w)], idx_all)

    gathers = [None, None]
    writes = [None, None]
    gathers[0] = pltpu.async_copy(
        table_hbm.at[idx_all.at[pl.ds(0, _CHUNK)]], rows0, gsem0)
    for j in range(nch):
        cur = j % 2
        nxt = 1 - cur
        if j + 1 < nch:
            if writes[nxt] is not None:
                writes[nxt].wait()
            gathers[nxt] = pltpu.async_copy(
                table_hbm.at[idx_all.at[pl.ds((j + 1) * _CHUNK, _CHUNK)]],
                rows[nxt], gsems[nxt])
        gathers[cur].wait()
        writes[cur] = pltpu.async_copy(
            rows[cur], out_hbm.at[pl.ds(base + j * _CHUNK, _CHUNK)],
            wsems[cur])
    writes[0].wait()
    writes[1].wait()


@functools.cache
def _gather_call(nrows):
    per_w = nrows // _NW
    return pl.kernel(
        functools.partial(_gather_body, per_w),
        out_type=jax.ShapeDtypeStruct((nrows, D), jnp.float32),
        mesh=plsc.VectorSubcoreMesh(core_axis_name="c", subcore_axis_name="s"),
        scratch_types=[
            pltpu.VMEM((per_w,), jnp.int32),
            pltpu.VMEM((_CHUNK, D), jnp.float32),
            pltpu.VMEM((_CHUNK, D), jnp.float32),
            pltpu.SemaphoreType.DMA,
            pltpu.SemaphoreType.DMA,
            pltpu.SemaphoreType.DMA,
            pltpu.SemaphoreType.DMA,
        ],
    )


def kernel(z_e, codebook):
    B, N, _ = z_e.shape
    z_flat = z_e.reshape(-1, D)
    a = jnp.sum(z_flat ** 2, axis=1, keepdims=True)
    b = jnp.sum(codebook ** 2, axis=1, keepdims=True).T

    codes2, loss_sum, ent_sum = _scores_call(a, z_flat, codebook, b, NROWS, 0)
    codes = codes2.reshape(-1)

    z_q = _gather_call(NROWS)(codes, codebook)

    z_q_st = z_q.reshape(B, N, D)
    codes_r = codes.reshape(B, N)
    loss_vq = (1.0 + BETA) * loss_sum[0, 0] / float(NROWS * D)
    perp = jnp.exp(ent_sum[0, 0] / float(NROWS))
    return (z_q_st, codes_r, loss_vq, perp)


# speedup vs baseline: 1.0708x; 1.0633x over previous
import functools

import jax
import jax.numpy as jnp
from jax import lax
from jax.experimental import pallas as pl
from jax.experimental.pallas import tpu as pltpu
from jax.experimental.pallas import tpu_sc as plsc

CB = 8192
D = 256
NROWS = 16384
BR = 512
BETA = 0.25


def _scores_body(a_ref, z_ref, c_hbm, b_ref, codes_ref, loss_ref, ent_ref,
                 c_vmem, csem):
    r = pl.program_id(0)

    @pl.when(r == 0)
    def _load_codebook():
        pltpu.async_copy(c_hbm, c_vmem, csem).wait()

    zc = lax.dot_general(z_ref[...], c_vmem[...], (((1,), (1,)), ((), ())),
                         preferred_element_type=jnp.float32)
    dist = (a_ref[...] - 2.0 * zc) + b_ref[...]

    m = jnp.min(dist, axis=1, keepdims=True)
    lanes = lax.broadcasted_iota(jnp.int32, dist.shape, 1).astype(jnp.float32)
    idx = jnp.min(jnp.where(dist == m, lanes, float(CB)), axis=1)
    codes_ref[...] = idx.reshape(BR, 1).astype(jnp.int32)

    e = jnp.exp(m - dist)
    s = jnp.sum(e, axis=1, keepdims=True)
    wd = jnp.sum(e * dist, axis=1, keepdims=True)
    ent = wd / s + (jnp.log(s) - m)

    loss_blk = jnp.sum(m)
    ent_blk = jnp.sum(ent)
    prev_l = jnp.where(r == 0, 0.0, loss_ref[0, 0])
    prev_e = jnp.where(r == 0, 0.0, ent_ref[0, 0])
    loss_ref[0, 0] = prev_l + loss_blk
    ent_ref[0, 0] = prev_e + ent_blk


def _scores_call(a, z_flat, codebook, b, nrows, off):
    grid = (nrows // BR,)
    ob = off // BR
    return pl.pallas_call(
        _scores_body,
        grid=grid,
        in_specs=[
            pl.BlockSpec((BR, 1), lambda r: (r + ob, 0)),
            pl.BlockSpec((BR, D), lambda r: (r + ob, 0)),
            pl.BlockSpec(memory_space=pltpu.MemorySpace.HBM),
            pl.BlockSpec((1, CB), lambda r: (0, 0)),
        ],
        scratch_shapes=[
            pltpu.VMEM((CB, D), jnp.float32),
            pltpu.SemaphoreType.DMA,
        ],
        out_specs=[
            pl.BlockSpec((BR, 1), lambda r: (r, 0)),
            pl.BlockSpec(memory_space=pltpu.SMEM),
            pl.BlockSpec(memory_space=pltpu.SMEM),
        ],
        out_shape=[
            jax.ShapeDtypeStruct((nrows, 1), jnp.int32),
            jax.ShapeDtypeStruct((1, 1), jnp.float32),
            jax.ShapeDtypeStruct((1, 1), jnp.float32),
        ],
    )(a, z_flat, codebook, b)


_NW = 32
_CHUNK = 128


def _gather_body(per_w, codes_hbm, table_hbm, out_hbm,
                 idx_all, rows0, rows1, gsem0, gsem1, wsem0, wsem1):
    nch = per_w // _CHUNK
    wid = lax.axis_index("s") * 2 + lax.axis_index("c")
    base = wid * per_w
    rows = (rows0, rows1)
    gsems = (gsem0, gsem1)
    wsems = (wsem0, wsem1)

    pltpu.sync_copy(codes_hbm.at[pl.ds(base, per_w)], idx_all)

    gathers = [None, None]
    writes = [None, None]
    gathers[0] = pltpu.async_copy(
        table_hbm.at[idx_all.at[pl.ds(0, _CHUNK)]], rows0, gsem0)
    for j in range(nch):
        cur = j % 2
        nxt = 1 - cur
        if j + 1 < nch:
            if writes[nxt] is not None:
                writes[nxt].wait()
            gathers[nxt] = pltpu.async_copy(
                table_hbm.at[idx_all.at[pl.ds((j + 1) * _CHUNK, _CHUNK)]],
                rows[nxt], gsems[nxt])
        gathers[cur].wait()
        writes[cur] = pltpu.async_copy(
            rows[cur], out_hbm.at[pl.ds(base + j * _CHUNK, _CHUNK)],
            wsems[cur])
    writes[0].wait()
    writes[1].wait()


@functools.cache
def _gather_call(nrows):
    per_w = nrows // _NW
    return pl.kernel(
        functools.partial(_gather_body, per_w),
        out_type=jax.ShapeDtypeStruct((nrows, D), jnp.float32),
        mesh=plsc.VectorSubcoreMesh(core_axis_name="c", subcore_axis_name="s"),
        scratch_types=[
            pltpu.VMEM((per_w,), jnp.int32),
            pltpu.VMEM((_CHUNK, D), jnp.float32),
            pltpu.VMEM((_CHUNK, D), jnp.float32),
            pltpu.SemaphoreType.DMA,
            pltpu.SemaphoreType.DMA,
            pltpu.SemaphoreType.DMA,
            pltpu.SemaphoreType.DMA,
        ],
    )


def kernel(z_e, codebook):
    B, N, _ = z_e.shape
    z_flat = z_e.reshape(-1, D)
    a = jnp.sum(z_flat ** 2, axis=1, keepdims=True)
    b = jnp.sum(codebook ** 2, axis=1, keepdims=True).T

    codes2, loss_sum, ent_sum = _scores_call(a, z_flat, codebook, b, NROWS, 0)
    codes = codes2.reshape(-1)

    z_q = _gather_call(NROWS)(codes, codebook)

    z_q_st = z_q.reshape(B, N, D)
    codes_r = codes.reshape(B, N)
    loss_vq = (1.0 + BETA) * loss_sum[0, 0] / float(NROWS * D)
    perp = jnp.exp(ent_sum[0, 0] / float(NROWS))
    return (z_q_st, codes_r, loss_vq, perp)
